# Initial kernel scaffold; baseline (speedup 1.0000x reference)
import functools

import jax
import jax.numpy as jnp
from jax import lax
from jax.experimental import pallas as pl
from jax.experimental.pallas import tpu as pltpu
from jax.experimental.pallas import tpu_sc as plsc

_NC = 2
_NS = 16
_L = 16
_CH = 128


def _node_slice(sid, n, fn):
  per = (-(-n // _NS) + 7) // 8 * 8
  last = n - per * (_NS - 1)
  assert last > 0

  @pl.when(sid < _NS - 1)
  def _():
    fn(sid * per, per)

  @pl.when(sid == _NS - 1)
  def _():
    fn(per * (_NS - 1), last)


def _make_sc_layer(n, e, d, with_deg):
  assert e % _CH == 0
  nq = e // _CH
  nw = _NC * _NS
  qper = nq // nw
  qrem = nq % nw

  mesh = plsc.VectorSubcoreMesh(
      core_axis_name="c", subcore_axis_name="s",
      num_cores=_NC, num_subcores=_NS)

  out_type = [
      jax.ShapeDtypeStruct((e,), jnp.float32),
      jax.ShapeDtypeStruct((_NC * n, d), jnp.float32),
      jax.ShapeDtypeStruct((_NC * n,), jnp.float32),
  ]
  if with_deg:
    out_type.append(jax.ShapeDtypeStruct((_NC * n,), jnp.float32))

  scratch = [
      pltpu.VMEM((n,), jnp.float32),
      pltpu.VMEM((n,), jnp.float32),
      pltpu.VMEM((_CH,), jnp.int32),
      pltpu.VMEM((_CH,), jnp.int32),
      pltpu.VMEM((_CH,), jnp.float32),
      pltpu.VMEM((_CH,), jnp.float32),
      pltpu.VMEM((_CH, d), jnp.float32),
      pltpu.VMEM_SHARED((n, d), jnp.float32),
      pltpu.VMEM_SHARED((n,), jnp.float32),
      pltpu.VMEM_SHARED((n,), jnp.float32),
      pltpu.SemaphoreType.DMA,
  ]

  def body(table_hbm, src_hbm, dst_hbm, sigl_hbm, sigr_hbm, z2_hbm, z1_hbm,
           *refs):
    if with_deg:
      (s_out, agg_out, t_out, deg_out, sigl_v, sigr_v, src_v, dst_v, s_v,
       ones_v, rows_v, agg_sh, t_sh, deg_sh, sem) = refs
    else:
      (s_out, agg_out, t_out, sigl_v, sigr_v, src_v, dst_v, s_v,
       ones_v, rows_v, agg_sh, t_sh, deg_sh, sem) = refs
      deg_out = None

    cid = lax.axis_index("c")
    sid = lax.axis_index("s")
    wid = cid * _NS + sid

    pltpu.sync_copy(sigl_hbm, sigl_v)
    pltpu.sync_copy(sigr_hbm, sigr_v)

    def zero2(off, size):
      pltpu.sync_copy(z2_hbm.at[pl.ds(off, size)], agg_sh.at[pl.ds(off, size)])

    def zero1(off, size):
      pltpu.sync_copy(z1_hbm.at[pl.ds(off, size)], t_sh.at[pl.ds(off, size)])
      if with_deg:
        pltpu.sync_copy(z1_hbm.at[pl.ds(off, size)],
                        deg_sh.at[pl.ds(off, size)])

    _node_slice(sid, n, zero2)
    _node_slice(sid, n, zero1)

    for k in range(_CH // _L):
      ones_v[pl.ds(k * _L, _L)] = jnp.full((_L,), 1.0, jnp.float32)

    plsc.subcore_barrier()

    start_q = wid * qper + jnp.minimum(wid, qrem)
    n_q = qper + jnp.where(wid < qrem, 1, 0) if qrem else qper

    def chunk(q, carry):
      base = (start_q + q) * _CH
      pltpu.sync_copy(src_hbm.at[pl.ds(base, _CH)], src_v)
      pltpu.sync_copy(dst_hbm.at[pl.ds(base, _CH)], dst_v)
      cp = pltpu.async_copy(table_hbm.at[src_v], rows_v, sem)

      def gate(j, c):
        sidx = src_v[pl.ds(j * _L, _L)]
        didx = dst_v[pl.ds(j * _L, _L)]
        z = plsc.load_gather(sigl_v, [sidx]) + plsc.load_gather(sigr_v, [didx])
        z = jnp.where(z >= 0, z, 0.2 * z)
        s_v[pl.ds(j * _L, _L)] = 1.0 / (1.0 + jnp.exp(-z))
        return c
      lax.fori_loop(0, _CH // _L, gate, 0)

      pltpu.sync_copy(s_v, s_out.at[pl.ds(base, _CH)])
      pltpu.sync_copy(s_v, t_sh.at[dst_v], add=True)
      if with_deg:
        pltpu.sync_copy(ones_v, deg_sh.at[dst_v], add=True)

      cp.wait()

      def scale(i, c):
        sc = s_v[i]
        for jj in range(d // _L):
          rows_v[i, pl.ds(jj * _L, _L)] *= sc
        return c
      lax.fori_loop(0, _CH, scale, 0)

      pltpu.sync_copy(rows_v, agg_sh.at[dst_v], add=True)
      return carry

    lax.fori_loop(0, n_q, chunk, 0)

    plsc.subcore_barrier()

    def wb2(off, size):
      pltpu.sync_copy(agg_sh.at[pl.ds(off, size)],
                      agg_out.at[pl.ds(cid * n + off, size)])

    def wb1(off, size):
      pltpu.sync_copy(t_sh.at[pl.ds(off, size)],
                      t_out.at[pl.ds(cid * n + off, size)])
      if with_deg:
        pltpu.sync_copy(deg_sh.at[pl.ds(off, size)],
                        deg_out.at[pl.ds(cid * n + off, size)])

    _node_slice(sid, n, wb2)
    _node_slice(sid, n, wb1)

  return pl.kernel(body, out_type=out_type, mesh=mesh, scratch_types=scratch)


@functools.lru_cache(maxsize=None)
def _sc_layer(n, e, d, with_deg):
  return _make_sc_layer(n, e, d, with_deg)


def _tc_pre(x, W1, att_l, att_r):
  n = x.shape[0]

  def body(x_ref, w1_ref, al_ref, ar_ref, sl_ref, sr_ref):
    wl = jnp.dot(al_ref[...], w1_ref[...], preferred_element_type=jnp.float32)
    wr = jnp.dot(ar_ref[...], w1_ref[...], preferred_element_type=jnp.float32)
    xx = x_ref[...]
    sl_ref[...] = jnp.sum(xx * wl, axis=1)
    sr_ref[...] = jnp.sum(xx * wr, axis=1)

  return pl.pallas_call(
      body,
      out_shape=(jax.ShapeDtypeStruct((n,), jnp.float32),
                 jax.ShapeDtypeStruct((n,), jnp.float32)),
  )(x, W1, att_l, att_r)


def _tc_combine1(x, aggp, tp, degp, WpT, bp, WnT, bn, WrT, W1n, aln, arn):
  n, f = x.shape

  def body(x_ref, aggp_ref, tp_ref, degp_ref, wp_ref, bp_ref, wn_ref, bn_ref,
           wr_ref, w1n_ref, aln_ref, arn_ref, h_ref, sl_ref, sr_ref):
    deg = degp_ref[pl.ds(0, n)] + degp_ref[pl.ds(n, n)]
    denom = jnp.maximum(deg, 1.0)
    agg = (aggp_ref[pl.ds(0, n), :] + aggp_ref[pl.ds(n, n), :]) / denom[:, None]
    t = tp_ref[pl.ds(0, n)] + tp_ref[pl.ds(n, n)]
    c = (deg - t) / denom
    xx = x_ref[...]
    outp = jnp.dot(agg, wp_ref[...], preferred_element_type=jnp.float32)
    outp += bp_ref[...]
    outn = jnp.dot(xx * c[:, None], wn_ref[...],
                   preferred_element_type=jnp.float32)
    outn += bn_ref[...]
    out = jnp.concatenate([outp, outn], axis=1)
    out += jnp.dot(xx, wr_ref[...], preferred_element_type=jnp.float32)
    h = jnp.maximum(out, 0.0)
    h_ref[...] = h
    wl = jnp.dot(aln_ref[...], w1n_ref[...], preferred_element_type=jnp.float32)
    wr2 = jnp.dot(arn_ref[...], w1n_ref[...],
                  preferred_element_type=jnp.float32)
    sl_ref[...] = jnp.sum(h * wl, axis=1)
    sr_ref[...] = jnp.sum(h * wr2, axis=1)

  return pl.pallas_call(
      body,
      out_shape=(jax.ShapeDtypeStruct((n, f), jnp.float32),
                 jax.ShapeDtypeStruct((n,), jnp.float32),
                 jax.ShapeDtypeStruct((n,), jnp.float32)),
  )(x, aggp, tp, degp, WpT, bp, WnT, bn, WrT, W1n, aln, arn)


def _tc_combine2(h, aggp, tp, degp, WpT, bp, WnT, bn, WrT):
  n, f = h.shape
  cout = WpT.shape[1] + WnT.shape[1]

  def body(h_ref, aggp_ref, tp_ref, degp_ref, wp_ref, bp_ref, wn_ref, bn_ref,
           wr_ref, out_ref):
    deg = degp_ref[pl.ds(0, n)] + degp_ref[pl.ds(n, n)]
    denom = jnp.maximum(deg, 1.0)
    agg = (aggp_ref[pl.ds(0, n), :] + aggp_ref[pl.ds(n, n), :]) / denom[:, None]
    t = tp_ref[pl.ds(0, n)] + tp_ref[pl.ds(n, n)]
    c = (deg - t) / denom
    hh = h_ref[...]
    outp = jnp.dot(agg, wp_ref[...], preferred_element_type=jnp.float32)
    outp += bp_ref[...]
    outn = jnp.dot(hh * c[:, None], wn_ref[...],
                   preferred_element_type=jnp.float32)
    outn += bn_ref[...]
    out = jnp.concatenate([outp, outn], axis=1)
    out += jnp.dot(hh, wr_ref[...], preferred_element_type=jnp.float32)
    m = jnp.max(out, axis=1, keepdims=True)
    ex = jnp.exp(out - m)
    lse = jnp.log(jnp.sum(ex, axis=1, keepdims=True)) + m
    out_ref[...] = out - lse

  return pl.pallas_call(
      body,
      out_shape=jax.ShapeDtypeStruct((n, cout), jnp.float32),
  )(h, aggp, tp, degp, WpT, bp, WnT, bn, WrT)


def kernel(x, edge_index, W1_1, att_l1, att_r1, Wp1, bp1, Wn1, bn1, Wr1,
           W1_2, att_l2, att_r2, Wp2, bp2, Wn2, bn2, Wr2):
  n, f = x.shape
  e = edge_index.shape[1]
  src = edge_index[0]
  dst = edge_index[1]
  z2 = jnp.zeros((n, f), jnp.float32)
  z1 = jnp.zeros((n,), jnp.float32)

  sl1, sr1 = _tc_pre(x, W1_1, att_l1, att_r1)
  s1, aggp1, tp1, degp1 = _sc_layer(n, e, f, True)(
      x, src, dst, sl1, sr1, z2, z1)
  h, sl2, sr2 = _tc_combine1(
      x, aggp1, tp1, degp1, Wp1.T, bp1, Wn1.T, bn1, Wr1.T,
      W1_2, att_l2, att_r2)
  s2, aggp2, tp2 = _sc_layer(n, e, f, False)(
      h, src, dst, sl2, sr2, z2, z1)
  logp = _tc_combine2(h, aggp2, tp2, degp1, Wp2.T, bp2, Wn2.T, bn2, Wr2.T)
  return (logp, s1, s2)


# speedup vs baseline: 19.6186x; 19.6186x over previous
import functools

import jax
import jax.numpy as jnp
from jax import lax
from jax.experimental import pallas as pl
from jax.experimental.pallas import tpu as pltpu
from jax.experimental.pallas import tpu_sc as plsc

_NC = 2
_NS = 16
_L = 16
_CH = 128


def _node_slice(sid, n, fn):
  per = (-(-n // _NS) + 7) // 8 * 8
  last = n - per * (_NS - 1)
  assert last > 0

  @pl.when(sid < _NS - 1)
  def _():
    fn(sid * per, per)

  @pl.when(sid == _NS - 1)
  def _():
    fn(per * (_NS - 1), last)


def _make_sc_layer(n, e, d, with_deg):
  assert e % _CH == 0
  nq = e // _CH
  nw = _NC * _NS
  qper = nq // nw
  qrem = nq % nw

  mesh = plsc.VectorSubcoreMesh(
      core_axis_name="c", subcore_axis_name="s",
      num_cores=_NC, num_subcores=_NS)

  out_type = [
      jax.ShapeDtypeStruct((e,), jnp.float32),
      jax.ShapeDtypeStruct((_NC * n, d), jnp.float32),
      jax.ShapeDtypeStruct((_NC * n,), jnp.float32),
  ]
  if with_deg:
    out_type.append(jax.ShapeDtypeStruct((_NC * n,), jnp.float32))

  scratch = [
      pltpu.VMEM((n,), jnp.float32),
      pltpu.VMEM((n,), jnp.float32),
      pltpu.VMEM((_CH,), jnp.int32),
      pltpu.VMEM((_CH,), jnp.int32),
      pltpu.VMEM((_CH,), jnp.float32),
      pltpu.VMEM((_CH,), jnp.float32),
      pltpu.VMEM((_CH, d), jnp.float32),
      pltpu.VMEM_SHARED((n, d), jnp.float32),
      pltpu.VMEM_SHARED((n,), jnp.float32),
      pltpu.VMEM_SHARED((n,), jnp.float32),
      pltpu.SemaphoreType.DMA,
  ]

  def body(table_hbm, src_hbm, dst_hbm, sigl_hbm, sigr_hbm, *refs):
    if with_deg:
      (s_out, agg_out, t_out, deg_out, sigl_v, sigr_v, src_v, dst_v, s_v,
       ones_v, rows_v, agg_sh, t_sh, deg_sh, sem) = refs
    else:
      (s_out, agg_out, t_out, sigl_v, sigr_v, src_v, dst_v, s_v,
       ones_v, rows_v, agg_sh, t_sh, deg_sh, sem) = refs
      deg_out = None

    cid = lax.axis_index("c")
    sid = lax.axis_index("s")
    wid = cid * _NS + sid

    pltpu.sync_copy(sigl_hbm, sigl_v)
    pltpu.sync_copy(sigr_hbm, sigr_v)

    for k in range(_CH // _L):
      ones_v[pl.ds(k * _L, _L)] = jnp.full((_L,), 1.0, jnp.float32)
      s_v[pl.ds(k * _L, _L)] = jnp.zeros((_L,), jnp.float32)

    def zrows(i, c):
      for jj in range(d // _L):
        rows_v[i, pl.ds(jj * _L, _L)] = jnp.zeros((_L,), jnp.float32)
      return c
    lax.fori_loop(0, _CH, zrows, 0)

    def zero(off, size):
      o = 0
      while o < size:
        sz = min(_CH, size - o)
        pltpu.sync_copy(rows_v.at[pl.ds(0, sz)],
                        agg_sh.at[pl.ds(off + o, sz)])
        pltpu.sync_copy(s_v.at[pl.ds(0, sz)], t_sh.at[pl.ds(off + o, sz)])
        if with_deg:
          pltpu.sync_copy(s_v.at[pl.ds(0, sz)], deg_sh.at[pl.ds(off + o, sz)])
        o += sz

    _node_slice(sid, n, zero)

    plsc.subcore_barrier()

    start_q = wid * qper + jnp.minimum(wid, qrem)
    n_q = qper + jnp.where(wid < qrem, 1, 0) if qrem else qper

    def chunk(q, carry):
      base = (start_q + q) * _CH
      pltpu.sync_copy(src_hbm.at[pl.ds(base, _CH)], src_v)
      pltpu.sync_copy(dst_hbm.at[pl.ds(base, _CH)], dst_v)
      cp = pltpu.async_copy(table_hbm.at[src_v], rows_v, sem)

      def gate(j, c):
        sidx = src_v[pl.ds(j * _L, _L)]
        didx = dst_v[pl.ds(j * _L, _L)]
        z = plsc.load_gather(sigl_v, [sidx]) + plsc.load_gather(sigr_v, [didx])
        z = jnp.where(z >= 0, z, 0.2 * z)
        s_v[pl.ds(j * _L, _L)] = 1.0 / (1.0 + jnp.exp(-z))
        return c
      lax.fori_loop(0, _CH // _L, gate, 0)

      pltpu.sync_copy(s_v, s_out.at[pl.ds(base, _CH)])
      pltpu.sync_copy(s_v, t_sh.at[dst_v], add=True)
      if with_deg:
        pltpu.sync_copy(ones_v, deg_sh.at[dst_v], add=True)

      cp.wait()

      def scale(g, c):
        sv = s_v[pl.ds(g * _L, _L)]
        for r in range(_L):
          sc = sv[r]
          for jj in range(d // _L):
            rows_v[g * _L + r, pl.ds(jj * _L, _L)] *= sc
        return c
      lax.fori_loop(0, _CH // _L, scale, 0)

      pltpu.sync_copy(rows_v, agg_sh.at[dst_v], add=True)
      return carry

    lax.fori_loop(0, n_q, chunk, 0)

    plsc.subcore_barrier()

    def wb(off, size):
      o = 0
      while o < size:
        sz = min(_CH, size - o)
        pltpu.sync_copy(agg_sh.at[pl.ds(off + o, sz)], rows_v.at[pl.ds(0, sz)])
        pltpu.sync_copy(rows_v.at[pl.ds(0, sz)],
                        agg_out.at[pl.ds(cid * n + off + o, sz)])
        pltpu.sync_copy(t_sh.at[pl.ds(off + o, sz)], s_v.at[pl.ds(0, sz)])
        pltpu.sync_copy(s_v.at[pl.ds(0, sz)],
                        t_out.at[pl.ds(cid * n + off + o, sz)])
        if with_deg:
          pltpu.sync_copy(deg_sh.at[pl.ds(off + o, sz)], s_v.at[pl.ds(0, sz)])
          pltpu.sync_copy(s_v.at[pl.ds(0, sz)],
                          deg_out.at[pl.ds(cid * n + off + o, sz)])
        o += sz

    _node_slice(sid, n, wb)

  return pl.kernel(
      body, out_type=out_type, mesh=mesh, scratch_types=scratch,
      compiler_params=pltpu.CompilerParams(needs_layout_passes=False))


@functools.lru_cache(maxsize=None)
def _sc_layer(n, e, d, with_deg):
  return _make_sc_layer(n, e, d, with_deg)


def _tc_pre(x, W1, att_l, att_r):
  n = x.shape[0]

  def body(x_ref, w1_ref, al_ref, ar_ref, sl_ref, sr_ref):
    wl = jnp.dot(al_ref[...], w1_ref[...], preferred_element_type=jnp.float32)
    wr = jnp.dot(ar_ref[...], w1_ref[...], preferred_element_type=jnp.float32)
    xx = x_ref[...]
    sl_ref[...] = jnp.sum(xx * wl, axis=1)
    sr_ref[...] = jnp.sum(xx * wr, axis=1)

  return pl.pallas_call(
      body,
      out_shape=(jax.ShapeDtypeStruct((n,), jnp.float32),
                 jax.ShapeDtypeStruct((n,), jnp.float32)),
  )(x, W1, att_l, att_r)


def _tc_combine1(x, aggp, tp, degp, WpT, bp, WnT, bn, WrT, W1n, aln, arn):
  n, f = x.shape

  def body(x_ref, aggp_ref, tp_ref, degp_ref, wp_ref, bp_ref, wn_ref, bn_ref,
           wr_ref, w1n_ref, aln_ref, arn_ref, h_ref, sl_ref, sr_ref):
    deg = degp_ref[pl.ds(0, n)] + degp_ref[pl.ds(n, n)]
    denom = jnp.maximum(deg, 1.0)
    agg = (aggp_ref[pl.ds(0, n), :] + aggp_ref[pl.ds(n, n), :]) / denom[:, None]
    t = tp_ref[pl.ds(0, n)] + tp_ref[pl.ds(n, n)]
    c = (deg - t) / denom
    xx = x_ref[...]
    outp = jnp.dot(agg, wp_ref[...], preferred_element_type=jnp.float32)
    outp += bp_ref[...]
    outn = jnp.dot(xx * c[:, None], wn_ref[...],
                   preferred_element_type=jnp.float32)
    outn += bn_ref[...]
    out = jnp.concatenate([outp, outn], axis=1)
    out += jnp.dot(xx, wr_ref[...], preferred_element_type=jnp.float32)
    h = jnp.maximum(out, 0.0)
    h_ref[...] = h
    wl = jnp.dot(aln_ref[...], w1n_ref[...], preferred_element_type=jnp.float32)
    wr2 = jnp.dot(arn_ref[...], w1n_ref[...],
                  preferred_element_type=jnp.float32)
    sl_ref[...] = jnp.sum(h * wl, axis=1)
    sr_ref[...] = jnp.sum(h * wr2, axis=1)

  return pl.pallas_call(
      body,
      out_shape=(jax.ShapeDtypeStruct((n, f), jnp.float32),
                 jax.ShapeDtypeStruct((n,), jnp.float32),
                 jax.ShapeDtypeStruct((n,), jnp.float32)),
  )(x, aggp, tp, degp, WpT, bp, WnT, bn, WrT, W1n, aln, arn)


def _tc_combine2(h, aggp, tp, degp, WpT, bp, WnT, bn, WrT):
  n, f = h.shape
  cout = WpT.shape[1] + WnT.shape[1]

  def body(h_ref, aggp_ref, tp_ref, degp_ref, wp_ref, bp_ref, wn_ref, bn_ref,
           wr_ref, out_ref):
    deg = degp_ref[pl.ds(0, n)] + degp_ref[pl.ds(n, n)]
    denom = jnp.maximum(deg, 1.0)
    agg = (aggp_ref[pl.ds(0, n), :] + aggp_ref[pl.ds(n, n), :]) / denom[:, None]
    t = tp_ref[pl.ds(0, n)] + tp_ref[pl.ds(n, n)]
    c = (deg - t) / denom
    hh = h_ref[...]
    outp = jnp.dot(agg, wp_ref[...], preferred_element_type=jnp.float32)
    outp += bp_ref[...]
    outn = jnp.dot(hh * c[:, None], wn_ref[...],
                   preferred_element_type=jnp.float32)
    outn += bn_ref[...]
    out = jnp.concatenate([outp, outn], axis=1)
    out += jnp.dot(hh, wr_ref[...], preferred_element_type=jnp.float32)
    m = jnp.max(out, axis=1, keepdims=True)
    ex = jnp.exp(out - m)
    lse = jnp.log(jnp.sum(ex, axis=1, keepdims=True)) + m
    out_ref[...] = out - lse

  return pl.pallas_call(
      body,
      out_shape=jax.ShapeDtypeStruct((n, cout), jnp.float32),
  )(h, aggp, tp, degp, WpT, bp, WnT, bn, WrT)


def kernel(x, edge_index, W1_1, att_l1, att_r1, Wp1, bp1, Wn1, bn1, Wr1,
           W1_2, att_l2, att_r2, Wp2, bp2, Wn2, bn2, Wr2):
  n, f = x.shape
  e = edge_index.shape[1]
  src = edge_index[0]
  dst = edge_index[1]

  sl1, sr1 = _tc_pre(x, W1_1, att_l1, att_r1)
  s1, aggp1, tp1, degp1 = _sc_layer(n, e, f, True)(
      x, src, dst, sl1, sr1)
  h, sl2, sr2 = _tc_combine1(
      x, aggp1, tp1, degp1, Wp1.T, bp1, Wn1.T, bn1, Wr1.T,
      W1_2, att_l2, att_r2)
  s2, aggp2, tp2 = _sc_layer(n, e, f, False)(
      h, src, dst, sl2, sr2)
  logp = _tc_combine2(h, aggp2, tp2, degp1, Wp2.T, bp2, Wn2.T, bn2, Wr2.T)
  return (logp, s1, s2)


# speedup vs baseline: 26.5941x; 1.3556x over previous
import functools

import jax
import jax.numpy as jnp
from jax import lax
from jax.experimental import pallas as pl
from jax.experimental.pallas import tpu as pltpu
from jax.experimental.pallas import tpu_sc as plsc

_NC = 2
_NS = 16
_L = 16
_CH = 80


def _node_slice(sid, n, fn):
  per = (-(-n // _NS) + 7) // 8 * 8
  last = n - per * (_NS - 1)
  assert last > 0

  @pl.when(sid < _NS - 1)
  def _():
    fn(sid * per, per)

  @pl.when(sid == _NS - 1)
  def _():
    fn(per * (_NS - 1), last)


def _make_sc_layer(n, e, d, with_deg):
  assert e % _CH == 0
  nq = e // _CH
  nw = _NC * _NS
  qper = nq // nw
  qrem = nq % nw

  mesh = plsc.VectorSubcoreMesh(
      core_axis_name="c", subcore_axis_name="s",
      num_cores=_NC, num_subcores=_NS)

  out_type = [
      jax.ShapeDtypeStruct((e,), jnp.float32),
      jax.ShapeDtypeStruct((_NC * n, d), jnp.float32),
      jax.ShapeDtypeStruct((_NC * n,), jnp.float32),
  ]
  if with_deg:
    out_type.append(jax.ShapeDtypeStruct((_NC * n,), jnp.float32))

  scratch = [
      pltpu.VMEM((n,), jnp.float32),
      pltpu.VMEM((n,), jnp.float32),
      pltpu.VMEM((_CH,), jnp.int32),
      pltpu.VMEM((_CH,), jnp.int32),
      pltpu.VMEM((_CH,), jnp.float32),
      pltpu.VMEM((_CH, d), jnp.float32),
      pltpu.VMEM((_CH,), jnp.int32),
      pltpu.VMEM((_CH,), jnp.int32),
      pltpu.VMEM((_CH,), jnp.float32),
      pltpu.VMEM((_CH, d), jnp.float32),
      pltpu.VMEM((_CH,), jnp.float32),
      pltpu.VMEM_SHARED((n, d), jnp.float32),
      pltpu.VMEM_SHARED((n,), jnp.float32),
      pltpu.VMEM_SHARED((n,), jnp.float32),
  ] + [pltpu.SemaphoreType.DMA] * 12

  def body(table_hbm, src_hbm, dst_hbm, sigl_hbm, sigr_hbm, *refs):
    (s_out, agg_out, t_out), refs = refs[:3], refs[3:]
    if with_deg:
      deg_out, refs = refs[0], refs[1:]
    else:
      deg_out = None
    (sigl_v, sigr_v, src_v0, dst_v0, s_v0, rows_v0, src_v1, dst_v1, s_v1,
     rows_v1, ones_v, agg_sh, t_sh, deg_sh, *sems) = refs
    src_v = (src_v0, src_v1)
    dst_v = (dst_v0, dst_v1)
    s_v = (s_v0, s_v1)
    rows_v = (rows_v0, rows_v1)
    sem_idx = sems[0:2]
    sem_g = sems[2:4]
    sem_s = sems[4:6]
    sem_t = sems[6:8]
    sem_d = sems[8:10]
    sem_r = sems[10:12]

    cid = lax.axis_index("c")
    sid = lax.axis_index("s")
    wid = cid * _NS + sid

    pltpu.sync_copy(sigl_hbm, sigl_v)
    pltpu.sync_copy(sigr_hbm, sigr_v)

    for k in range(_CH // _L):
      ones_v[pl.ds(k * _L, _L)] = jnp.full((_L,), 1.0, jnp.float32)
      s_v0[pl.ds(k * _L, _L)] = jnp.zeros((_L,), jnp.float32)

    def zrows(i, c):
      for jj in range(d // _L):
        rows_v0[i, pl.ds(jj * _L, _L)] = jnp.zeros((_L,), jnp.float32)
      return c
    lax.fori_loop(0, _CH, zrows, 0)

    def zero(off, size):
      o = 0
      while o < size:
        sz = min(_CH, size - o)
        pltpu.sync_copy(rows_v0.at[pl.ds(0, sz)],
                        agg_sh.at[pl.ds(off + o, sz)])
        pltpu.sync_copy(s_v0.at[pl.ds(0, sz)], t_sh.at[pl.ds(off + o, sz)])
        if with_deg:
          pltpu.sync_copy(s_v0.at[pl.ds(0, sz)],
                          deg_sh.at[pl.ds(off + o, sz)])
        o += sz

    _node_slice(sid, n, zero)

    plsc.subcore_barrier()

    start_q = wid * qper + jnp.minimum(wid, qrem)
    n_q = qper + jnp.where(wid < qrem, 1, 0) if qrem else qper

    def idx_fetch(q, b):
      base = (start_q + q) * _CH
      pltpu.async_copy(src_hbm.at[pl.ds(base, _CH)], src_v[b], sem_idx[b])
      pltpu.async_copy(dst_hbm.at[pl.ds(base, _CH)], dst_v[b], sem_idx[b])

    idx_fetch(0, 0)

    def chunk_step(q, b):
      o = 1 - b
      base = (start_q + q) * _CH
      pltpu.make_async_copy(src_hbm.at[pl.ds(base, _CH)], src_v[b],
                            sem_idx[b]).wait()
      pltpu.make_async_copy(dst_hbm.at[pl.ds(base, _CH)], dst_v[b],
                            sem_idx[b]).wait()
      gcp = pltpu.async_copy(table_hbm.at[src_v[b]], rows_v[b], sem_g[b])

      def gate(j, c):
        sidx = src_v[b][pl.ds(j * _L, _L)]
        didx = dst_v[b][pl.ds(j * _L, _L)]
        z = plsc.load_gather(sigl_v, [sidx]) + plsc.load_gather(sigr_v, [didx])
        z = jnp.where(z >= 0, z, 0.2 * z)
        s_v[b][pl.ds(j * _L, _L)] = 1.0 / (1.0 + jnp.exp(-z))
        return c
      lax.fori_loop(0, _CH // _L, gate, 0)

      w_s = pltpu.async_copy(s_v[b], s_out.at[pl.ds(base, _CH)], sem_s[b])
      w_t = pltpu.async_copy(s_v[b], t_sh.at[dst_v[b]], sem_t[b], add=True)
      if with_deg:
        w_d = pltpu.async_copy(ones_v, deg_sh.at[dst_v[b]], sem_d[b], add=True)

      @pl.when(q >= 1)
      def _():
        pltpu.make_async_copy(rows_v[o], agg_sh.at[dst_v[o]],
                              sem_r[o]).wait()

      @pl.when(q + 1 < n_q)
      def _():
        pltpu.async_copy(src_hbm.at[pl.ds(base + _CH, _CH)], src_v[o],
                         sem_idx[o])
        pltpu.async_copy(dst_hbm.at[pl.ds(base + _CH, _CH)], dst_v[o],
                         sem_idx[o])

      gcp.wait()

      def scale(g, c):
        sv = s_v[b][pl.ds(g * _L, _L)]
        for r in range(_L):
          sc = sv[r]
          for jj in range(d // _L):
            rows_v[b][g * _L + r, pl.ds(jj * _L, _L)] *= sc
        return c
      lax.fori_loop(0, _CH // _L, scale, 0)

      pltpu.async_copy(rows_v[b], agg_sh.at[dst_v[b]], sem_r[b], add=True)
      w_s.wait()
      w_t.wait()
      if with_deg:
        w_d.wait()

    def pair(qq, carry):
      for b in range(2):
        q = qq * 2 + b

        @pl.when(q < n_q)
        def _():
          chunk_step(q, b)
      return carry

    lax.fori_loop(0, (n_q + 1) // 2, pair, 0)

    def drain_last(b):
      pltpu.make_async_copy(rows_v[b], agg_sh.at[dst_v[b]], sem_r[b]).wait()

    if isinstance(n_q, int):
      drain_last((n_q - 1) % 2)
    else:
      @pl.when((n_q - 1) % 2 == 0)
      def _():
        drain_last(0)

      @pl.when((n_q - 1) % 2 == 1)
      def _():
        drain_last(1)

    plsc.subcore_barrier()

    def wb(off, size):
      o = 0
      while o < size:
        sz = min(_CH, size - o)
        pltpu.sync_copy(agg_sh.at[pl.ds(off + o, sz)],
                        rows_v0.at[pl.ds(0, sz)])
        pltpu.sync_copy(rows_v0.at[pl.ds(0, sz)],
                        agg_out.at[pl.ds(cid * n + off + o, sz)])
        pltpu.sync_copy(t_sh.at[pl.ds(off + o, sz)], s_v0.at[pl.ds(0, sz)])
        pltpu.sync_copy(s_v0.at[pl.ds(0, sz)],
                        t_out.at[pl.ds(cid * n + off + o, sz)])
        if with_deg:
          pltpu.sync_copy(deg_sh.at[pl.ds(off + o, sz)],
                          s_v0.at[pl.ds(0, sz)])
          pltpu.sync_copy(s_v0.at[pl.ds(0, sz)],
                          deg_out.at[pl.ds(cid * n + off + o, sz)])
        o += sz

    _node_slice(sid, n, wb)

  return pl.kernel(
      body, out_type=out_type, mesh=mesh, scratch_types=scratch,
      compiler_params=pltpu.CompilerParams(needs_layout_passes=False))


@functools.lru_cache(maxsize=None)
def _sc_layer(n, e, d, with_deg):
  return _make_sc_layer(n, e, d, with_deg)


def _tc_pre(x, W1, att_l, att_r):
  n = x.shape[0]

  def body(x_ref, w1_ref, al_ref, ar_ref, sl_ref, sr_ref):
    wl = jnp.dot(al_ref[...], w1_ref[...], preferred_element_type=jnp.float32)
    wr = jnp.dot(ar_ref[...], w1_ref[...], preferred_element_type=jnp.float32)
    xx = x_ref[...]
    sl_ref[...] = jnp.sum(xx * wl, axis=1)
    sr_ref[...] = jnp.sum(xx * wr, axis=1)

  return pl.pallas_call(
      body,
      out_shape=(jax.ShapeDtypeStruct((n,), jnp.float32),
                 jax.ShapeDtypeStruct((n,), jnp.float32)),
  )(x, W1, att_l, att_r)


def _tc_combine1(x, aggp, tp, degp, WpT, bp, WnT, bn, WrT, W1n, aln, arn):
  n, f = x.shape

  def body(x_ref, aggp_ref, tp_ref, degp_ref, wp_ref, bp_ref, wn_ref, bn_ref,
           wr_ref, w1n_ref, aln_ref, arn_ref, h_ref, sl_ref, sr_ref):
    deg = degp_ref[pl.ds(0, n)] + degp_ref[pl.ds(n, n)]
    denom = jnp.maximum(deg, 1.0)
    agg = (aggp_ref[pl.ds(0, n), :] + aggp_ref[pl.ds(n, n), :]) / denom[:, None]
    t = tp_ref[pl.ds(0, n)] + tp_ref[pl.ds(n, n)]
    c = (deg - t) / denom
    xx = x_ref[...]
    outp = jnp.dot(agg, wp_ref[...], preferred_element_type=jnp.float32)
    outp += bp_ref[...]
    outn = jnp.dot(xx * c[:, None], wn_ref[...],
                   preferred_element_type=jnp.float32)
    outn += bn_ref[...]
    out = jnp.concatenate([outp, outn], axis=1)
    out += jnp.dot(xx, wr_ref[...], preferred_element_type=jnp.float32)
    h = jnp.maximum(out, 0.0)
    h_ref[...] = h
    wl = jnp.dot(aln_ref[...], w1n_ref[...], preferred_element_type=jnp.float32)
    wr2 = jnp.dot(arn_ref[...], w1n_ref[...],
                  preferred_element_type=jnp.float32)
    sl_ref[...] = jnp.sum(h * wl, axis=1)
    sr_ref[...] = jnp.sum(h * wr2, axis=1)

  return pl.pallas_call(
      body,
      out_shape=(jax.ShapeDtypeStruct((n, f), jnp.float32),
                 jax.ShapeDtypeStruct((n,), jnp.float32),
                 jax.ShapeDtypeStruct((n,), jnp.float32)),
  )(x, aggp, tp, degp, WpT, bp, WnT, bn, WrT, W1n, aln, arn)


def _tc_combine2(h, aggp, tp, degp, WpT, bp, WnT, bn, WrT):
  n, f = h.shape
  cout = WpT.shape[1] + WnT.shape[1]

  def body(h_ref, aggp_ref, tp_ref, degp_ref, wp_ref, bp_ref, wn_ref, bn_ref,
           wr_ref, out_ref):
    deg = degp_ref[pl.ds(0, n)] + degp_ref[pl.ds(n, n)]
    denom = jnp.maximum(deg, 1.0)
    agg = (aggp_ref[pl.ds(0, n), :] + aggp_ref[pl.ds(n, n), :]) / denom[:, None]
    t = tp_ref[pl.ds(0, n)] + tp_ref[pl.ds(n, n)]
    c = (deg - t) / denom
    hh = h_ref[...]
    outp = jnp.dot(agg, wp_ref[...], preferred_element_type=jnp.float32)
    outp += bp_ref[...]
    outn = jnp.dot(hh * c[:, None], wn_ref[...],
                   preferred_element_type=jnp.float32)
    outn += bn_ref[...]
    out = jnp.concatenate([outp, outn], axis=1)
    out += jnp.dot(hh, wr_ref[...], preferred_element_type=jnp.float32)
    m = jnp.max(out, axis=1, keepdims=True)
    ex = jnp.exp(out - m)
    lse = jnp.log(jnp.sum(ex, axis=1, keepdims=True)) + m
    out_ref[...] = out - lse

  return pl.pallas_call(
      body,
      out_shape=jax.ShapeDtypeStruct((n, cout), jnp.float32),
  )(h, aggp, tp, degp, WpT, bp, WnT, bn, WrT)


def kernel(x, edge_index, W1_1, att_l1, att_r1, Wp1, bp1, Wn1, bn1, Wr1,
           W1_2, att_l2, att_r2, Wp2, bp2, Wn2, bn2, Wr2):
  n, f = x.shape
  e = edge_index.shape[1]
  src = edge_index[0]
  dst = edge_index[1]

  sl1, sr1 = _tc_pre(x, W1_1, att_l1, att_r1)
  s1, aggp1, tp1, degp1 = _sc_layer(n, e, f, True)(
      x, src, dst, sl1, sr1)
  h, sl2, sr2 = _tc_combine1(
      x, aggp1, tp1, degp1, Wp1.T, bp1, Wn1.T, bn1, Wr1.T,
      W1_2, att_l2, att_r2)
  s2, aggp2, tp2 = _sc_layer(n, e, f, False)(
      h, src, dst, sl2, sr2)
  logp = _tc_combine2(h, aggp2, tp2, degp1, Wp2.T, bp2, Wn2.T, bn2, Wr2.T)
  return (logp, s1, s2)


# speedup vs baseline: 29.8431x; 1.1222x over previous
import functools

import jax
import jax.numpy as jnp
from jax import lax
from jax.experimental import pallas as pl
from jax.experimental.pallas import tpu as pltpu
from jax.experimental.pallas import tpu_sc as plsc

_NC = 2
_NS = 16
_L = 16
_CH = 80
_F = 128


def _node_slice(sid, n, fn):
  per = (-(-n // _NS) + 7) // 8 * 8
  last = n - per * (_NS - 1)
  assert last > 0

  @pl.when(sid < _NS - 1)
  def _():
    fn(sid * per, per)

  @pl.when(sid == _NS - 1)
  def _():
    fn(per * (_NS - 1), last)


def _make_sc_layer(n, e, d, with_deg):
  assert e % _CH == 0
  nq = e // _CH
  nw = _NC * _NS
  qper = nq // nw
  qrem = nq % nw

  mesh = plsc.VectorSubcoreMesh(
      core_axis_name="c", subcore_axis_name="s",
      num_cores=_NC, num_subcores=_NS)

  out_type = [
      jax.ShapeDtypeStruct((e,), jnp.float32),
      jax.ShapeDtypeStruct((_NC * n, _F), jnp.float32),
      jax.ShapeDtypeStruct((_NC * n,), jnp.float32),
  ]
  if with_deg:
    out_type.append(jax.ShapeDtypeStruct((_NC * n,), jnp.float32))

  scratch = [
      pltpu.VMEM((n,), jnp.float32),
      pltpu.VMEM((n,), jnp.float32),
      pltpu.VMEM((_CH,), jnp.int32),
      pltpu.VMEM((_CH,), jnp.int32),
      pltpu.VMEM((_CH,), jnp.float32),
      pltpu.VMEM((_CH, _F), jnp.float32),
      pltpu.VMEM((_CH,), jnp.int32),
      pltpu.VMEM((_CH,), jnp.int32),
      pltpu.VMEM((_CH,), jnp.float32),
      pltpu.VMEM((_CH, _F), jnp.float32),
      pltpu.VMEM((_CH,), jnp.float32),
      pltpu.VMEM_SHARED((n, _F), jnp.float32),
      pltpu.VMEM_SHARED((n,), jnp.float32),
      pltpu.VMEM_SHARED((n,), jnp.float32),
  ] + [pltpu.SemaphoreType.DMA] * 12

  def body(table_hbm, src_hbm, dst_hbm, sigl_hbm, sigr_hbm, *refs):
    (s_out, agg_out, t_out), refs = refs[:3], refs[3:]
    if with_deg:
      deg_out, refs = refs[0], refs[1:]
    else:
      deg_out = None
    (sigl_v, sigr_v, src_v0, dst_v0, s_v0, rows_v0, src_v1, dst_v1,
     s_v1, rows_v1, ones_v, agg_sh, t_sh, deg_sh, *sems) = refs
    src_v = (src_v0, src_v1)
    dst_v = (dst_v0, dst_v1)
    s_v = (s_v0, s_v1)
    rows_v = (rows_v0, rows_v1)
    sem_idx = sems[0:2]
    sem_g = sems[2:4]
    sem_s = sems[4:6]
    sem_t = sems[6:8]
    sem_d = sems[8:10]
    sem_r = sems[10:12]

    cid = lax.axis_index("c")
    sid = lax.axis_index("s")
    wid = cid * _NS + sid

    pltpu.sync_copy(sigl_hbm, sigl_v)
    pltpu.sync_copy(sigr_hbm, sigr_v)

    for k in range(_CH // _L):
      ones_v[pl.ds(k * _L, _L)] = jnp.full((_L,), 1.0, jnp.float32)
      s_v0[pl.ds(k * _L, _L)] = jnp.zeros((_L,), jnp.float32)

    def zrows(i, c):
      for jj in range(_F // _L):
        rows_v0[i, pl.ds(jj * _L, _L)] = jnp.zeros((_L,), jnp.float32)
      return c
    lax.fori_loop(0, _CH, zrows, 0)

    def zero(off, size):
      o = 0
      while o < size:
        sz = min(_CH, size - o)
        pltpu.sync_copy(rows_v0.at[pl.ds(0, sz)],
                        agg_sh.at[pl.ds(off + o, sz)])
        pltpu.sync_copy(s_v0.at[pl.ds(0, sz)], t_sh.at[pl.ds(off + o, sz)])
        if with_deg:
          pltpu.sync_copy(s_v0.at[pl.ds(0, sz)],
                          deg_sh.at[pl.ds(off + o, sz)])
        o += sz

    _node_slice(sid, n, zero)

    plsc.subcore_barrier()

    start_q = wid * qper + jnp.minimum(wid, qrem)
    n_q = qper + jnp.where(wid < qrem, 1, 0) if qrem else qper

    def idx_fetch(q, b):
      base = (start_q + q) * _CH
      pltpu.async_copy(src_hbm.at[pl.ds(base, _CH)], src_v[b], sem_idx[b])
      pltpu.async_copy(dst_hbm.at[pl.ds(base, _CH)], dst_v[b], sem_idx[b])

    idx_fetch(0, 0)

    def chunk_step(q, b):
      o = 1 - b
      base = (start_q + q) * _CH
      pltpu.make_async_copy(src_hbm.at[pl.ds(base, _CH)], src_v[b],
                            sem_idx[b]).wait()
      pltpu.make_async_copy(dst_hbm.at[pl.ds(base, _CH)], dst_v[b],
                            sem_idx[b]).wait()
      gcp = pltpu.async_copy(table_hbm.at[src_v[b]], rows_v[b], sem_g[b])

      def gate(j, c):
        sidx = src_v[b][pl.ds(j * _L, _L)]
        didx = dst_v[b][pl.ds(j * _L, _L)]
        z = plsc.load_gather(sigl_v, [sidx]) + plsc.load_gather(sigr_v, [didx])
        z = jnp.where(z >= 0, z, 0.2 * z)
        s_v[b][pl.ds(j * _L, _L)] = 1.0 / (1.0 + jnp.exp(-z))
        return c
      lax.fori_loop(0, _CH // _L, gate, 0)

      w_s = pltpu.async_copy(s_v[b], s_out.at[pl.ds(base, _CH)], sem_s[b])
      w_t = pltpu.async_copy(s_v[b], t_sh.at[dst_v[b]], sem_t[b], add=True)
      if with_deg:
        w_d = pltpu.async_copy(ones_v, deg_sh.at[dst_v[b]], sem_d[b], add=True)

      @pl.when(q >= 1)
      def _():
        pltpu.make_async_copy(rows_v[o], agg_sh.at[dst_v[o]],
                              sem_r[o]).wait()

      @pl.when(q + 1 < n_q)
      def _():
        pltpu.async_copy(src_hbm.at[pl.ds(base + _CH, _CH)], src_v[o],
                         sem_idx[o])
        pltpu.async_copy(dst_hbm.at[pl.ds(base + _CH, _CH)], dst_v[o],
                         sem_idx[o])

      gcp.wait()

      def scale(g, c):
        sv = s_v[b][pl.ds(g * _L, _L)]
        for r in range(_L):
          sc = sv[r]
          for jj in range(d // _L):
            rows_v[b][g * _L + r, pl.ds(jj * _L, _L)] *= sc
        return c
      lax.fori_loop(0, _CH // _L, scale, 0)

      pltpu.async_copy(rows_v[b], agg_sh.at[dst_v[b]], sem_r[b], add=True)
      w_s.wait()
      w_t.wait()
      if with_deg:
        w_d.wait()

    def pair(qq, carry):
      for b in range(2):
        q = qq * 2 + b

        @pl.when(q < n_q)
        def _():
          chunk_step(q, b)
      return carry

    lax.fori_loop(0, (n_q + 1) // 2, pair, 0)

    def drain_last(b):
      pltpu.make_async_copy(rows_v[b], agg_sh.at[dst_v[b]], sem_r[b]).wait()

    if isinstance(n_q, int):
      drain_last((n_q - 1) % 2)
    else:
      @pl.when((n_q - 1) % 2 == 0)
      def _():
        drain_last(0)

      @pl.when((n_q - 1) % 2 == 1)
      def _():
        drain_last(1)

    plsc.subcore_barrier()

    def wb(off, size):
      o = 0
      while o < size:
        sz = min(_CH, size - o)
        pltpu.sync_copy(agg_sh.at[pl.ds(off + o, sz)],
                        rows_v0.at[pl.ds(0, sz)])
        pltpu.sync_copy(rows_v0.at[pl.ds(0, sz)],
                        agg_out.at[pl.ds(cid * n + off + o, sz)])
        pltpu.sync_copy(t_sh.at[pl.ds(off + o, sz)], s_v0.at[pl.ds(0, sz)])
        pltpu.sync_copy(s_v0.at[pl.ds(0, sz)],
                        t_out.at[pl.ds(cid * n + off + o, sz)])
        if with_deg:
          pltpu.sync_copy(deg_sh.at[pl.ds(off + o, sz)],
                          s_v0.at[pl.ds(0, sz)])
          pltpu.sync_copy(s_v0.at[pl.ds(0, sz)],
                          deg_out.at[pl.ds(cid * n + off + o, sz)])
        o += sz

    _node_slice(sid, n, wb)

  return pl.kernel(
      body, out_type=out_type, mesh=mesh, scratch_types=scratch,
      compiler_params=pltpu.CompilerParams(needs_layout_passes=False))


@functools.lru_cache(maxsize=None)
def _sc_layer(n, e, d, with_deg):
  return _make_sc_layer(n, e, d, with_deg)


def _tc_pre(x, W1, att_l, att_r, WpT):
  n = x.shape[0]
  dp = WpT.shape[1]

  def body(x_ref, w1_ref, al_ref, ar_ref, wp_ref, sl_ref, sr_ref, y_ref):
    wl = jnp.dot(al_ref[...], w1_ref[...], preferred_element_type=jnp.float32)
    wr = jnp.dot(ar_ref[...], w1_ref[...], preferred_element_type=jnp.float32)
    xx = x_ref[...]
    sl_ref[...] = jnp.sum(xx * wl, axis=1)
    sr_ref[...] = jnp.sum(xx * wr, axis=1)
    y = jnp.dot(xx, wp_ref[...], preferred_element_type=jnp.float32)
    y_ref[...] = jnp.pad(y, ((0, 0), (0, _F - dp)))

  return pl.pallas_call(
      body,
      out_shape=(jax.ShapeDtypeStruct((n,), jnp.float32),
                 jax.ShapeDtypeStruct((n,), jnp.float32),
                 jax.ShapeDtypeStruct((n, _F), jnp.float32)),
  )(x, W1, att_l, att_r, WpT)


def _tc_combine1(x, aggp, tp, degp, bp, WnT, bn, WrT, W1n, aln, arn, WpnT):
  n, f = x.shape
  dp = bp.shape[0]
  dpn = WpnT.shape[1]

  def body(x_ref, aggp_ref, tp_ref, degp_ref, bp_ref, wn_ref, bn_ref,
           wr_ref, w1n_ref, aln_ref, arn_ref, wpn_ref,
           h_ref, sl_ref, sr_ref, y_ref):
    deg = degp_ref[pl.ds(0, n)] + degp_ref[pl.ds(n, n)]
    denom = jnp.maximum(deg, 1.0)
    t = tp_ref[pl.ds(0, n)] + tp_ref[pl.ds(n, n)]
    c = (deg - t) / denom
    xx = x_ref[...]
    aggs = aggp_ref[pl.ds(0, n), :] + aggp_ref[pl.ds(n, n), :]
    outp = aggs[:, :dp] / denom[:, None]
    outp += bp_ref[...]
    outn = jnp.dot(xx * c[:, None], wn_ref[...],
                   preferred_element_type=jnp.float32)
    outn += bn_ref[...]
    out = jnp.concatenate([outp, outn], axis=1)
    out += jnp.dot(xx, wr_ref[...], preferred_element_type=jnp.float32)
    h = jnp.maximum(out, 0.0)
    h_ref[...] = h
    wl = jnp.dot(aln_ref[...], w1n_ref[...], preferred_element_type=jnp.float32)
    wr2 = jnp.dot(arn_ref[...], w1n_ref[...],
                  preferred_element_type=jnp.float32)
    sl_ref[...] = jnp.sum(h * wl, axis=1)
    sr_ref[...] = jnp.sum(h * wr2, axis=1)
    y = jnp.dot(h, wpn_ref[...], preferred_element_type=jnp.float32)
    y_ref[...] = jnp.pad(y, ((0, 0), (0, _F - dpn)))

  return pl.pallas_call(
      body,
      out_shape=(jax.ShapeDtypeStruct((n, 2 * dp), jnp.float32),
                 jax.ShapeDtypeStruct((n,), jnp.float32),
                 jax.ShapeDtypeStruct((n,), jnp.float32),
                 jax.ShapeDtypeStruct((n, _F), jnp.float32)),
  )(x, aggp, tp, degp, bp, WnT, bn, WrT, W1n, aln, arn, WpnT)


def _tc_combine2(h, aggp, tp, degp, bp, WnT, bn, WrT):
  n, f = h.shape
  dp = bp.shape[0]

  def body(h_ref, aggp_ref, tp_ref, degp_ref, bp_ref, wn_ref, bn_ref,
           wr_ref, out_ref):
    deg = degp_ref[pl.ds(0, n)] + degp_ref[pl.ds(n, n)]
    denom = jnp.maximum(deg, 1.0)
    t = tp_ref[pl.ds(0, n)] + tp_ref[pl.ds(n, n)]
    c = (deg - t) / denom
    hh = h_ref[...]
    aggs = aggp_ref[pl.ds(0, n), :] + aggp_ref[pl.ds(n, n), :]
    outp = aggs[:, :dp] / denom[:, None]
    outp += bp_ref[...]
    outn = jnp.dot(hh * c[:, None], wn_ref[...],
                   preferred_element_type=jnp.float32)
    outn += bn_ref[...]
    out = jnp.concatenate([outp, outn], axis=1)
    out += jnp.dot(hh, wr_ref[...], preferred_element_type=jnp.float32)
    m = jnp.max(out, axis=1, keepdims=True)
    ex = jnp.exp(out - m)
    lse = jnp.log(jnp.sum(ex, axis=1, keepdims=True)) + m
    out_ref[...] = out - lse

  return pl.pallas_call(
      body,
      out_shape=jax.ShapeDtypeStruct((n, 2 * dp), jnp.float32),
  )(h, aggp, tp, degp, bp, WnT, bn, WrT)


def kernel(x, edge_index, W1_1, att_l1, att_r1, Wp1, bp1, Wn1, bn1, Wr1,
           W1_2, att_l2, att_r2, Wp2, bp2, Wn2, bn2, Wr2):
  n, f = x.shape
  e = edge_index.shape[1]
  src = edge_index[0]
  dst = edge_index[1]
  dp1 = Wp1.shape[0]
  dp2 = Wp2.shape[0]

  sl1, sr1, y1 = _tc_pre(x, W1_1, att_l1, att_r1, Wp1.T)
  s1, aggp1, tp1, degp1 = _sc_layer(n, e, dp1, True)(
      y1, src, dst, sl1, sr1)
  h, sl2, sr2, y2 = _tc_combine1(
      x, aggp1, tp1, degp1, bp1, Wn1.T, bn1, Wr1.T,
      W1_2, att_l2, att_r2, Wp2.T)
  s2, aggp2, tp2 = _sc_layer(n, e, dp2, False)(
      y2, src, dst, sl2, sr2)
  logp = _tc_combine2(h, aggp2, tp2, degp1, bp2, Wn2.T, bn2, Wr2.T)
  return (logp, s1, s2)
